# baseline (speedup 1.0000x reference)
import functools

import jax
import jax.numpy as jnp
from jax import lax
from jax.experimental import pallas as pl
from jax.experimental.pallas import tpu as pltpu
from jax.experimental.pallas import tpu_sc as plsc

LATENT = 64
BATCH = 16384

_info = plsc.get_sparse_core_info()
_NC = _info.num_cores
_NS = _info.num_subcores
_L = _info.num_lanes
_NW = _NC * _NS
_BPW = BATCH // _NW
_CHUNK = 128
_NCHUNK = _BPW // _CHUNK

_mesh = plsc.VectorSubcoreMesh(core_axis_name="c", subcore_axis_name="s")


@functools.partial(
    pl.kernel,
    out_type=jax.ShapeDtypeStruct((BATCH,), jnp.float32),
    mesh=_mesh,
    compiler_params=pltpu.CompilerParams(
        needs_layout_passes=False, use_tc_tiling_on_sc=False),
    scratch_types=[
        pltpu.VMEM((_BPW,), jnp.int32),
        pltpu.VMEM((_BPW,), jnp.int32),
        pltpu.VMEM((_BPW, LATENT), jnp.float32),
        pltpu.VMEM((_BPW, LATENT), jnp.float32),
        pltpu.VMEM((_BPW,), jnp.float32),
        pltpu.VMEM((_BPW,), jnp.float32),
        pltpu.VMEM((_L,), jnp.float32),
        pltpu.VMEM((_BPW,), jnp.float32),
        pltpu.SemaphoreType.DMA,
    ],
)
def _sc_scores(uid_hbm, iid_hbm, utab_hbm, itab_hbm, ubias_hbm, ibias_hbm,
               gb_hbm, out_hbm,
               uid_v, iid_v, urows_v, irows_v, ubias_v, ibias_v, gb_v,
               out_v, sem):
    wid = lax.axis_index("s") * _NC + lax.axis_index("c")
    base = wid * _BPW

    pltpu.sync_copy(uid_hbm.at[pl.ds(base, _BPW)], uid_v)
    pltpu.sync_copy(iid_hbm.at[pl.ds(base, _BPW)], iid_v)
    pltpu.sync_copy(gb_hbm, gb_v)

    copies = []
    for k in range(_NCHUNK):
        sl = pl.ds(k * _CHUNK, _CHUNK)
        copies.append(pltpu.async_copy(
            utab_hbm.at[uid_v.at[sl]], urows_v.at[sl], sem))
        copies.append(pltpu.async_copy(
            itab_hbm.at[iid_v.at[sl]], irows_v.at[sl], sem))
        copies.append(pltpu.async_copy(
            ubias_hbm.at[uid_v.at[sl]], ubias_v.at[sl], sem))
        copies.append(pltpu.async_copy(
            ibias_hbm.at[iid_v.at[sl]], ibias_v.at[sl], sem))
    for c in copies:
        c.wait()

    gb = gb_v[...]
    lane = lax.iota(jnp.int32, _L)

    def group(g, carry):
        ridx = g * _L + lane
        accs = [jnp.zeros((_L,), jnp.float32) for _ in range(4)]
        for d in range(LATENT):
            cidx = jnp.full((_L,), d, jnp.int32)
            gu = plsc.load_gather(urows_v, [ridx, cidx])
            gi = plsc.load_gather(irows_v, [ridx, cidx])
            accs[d % 4] = accs[d % 4] + gu * gi
        res = (accs[0] + accs[1]) + (accs[2] + accs[3])
        res = res + ubias_v[pl.ds(g * _L, _L)] + ibias_v[pl.ds(g * _L, _L)] + gb
        out_v[pl.ds(g * _L, _L)] = res
        return carry

    lax.fori_loop(0, _BPW // _L, group, 0)

    pltpu.sync_copy(out_v, out_hbm.at[pl.ds(base, _BPW)])


def kernel(user_ids, item_ids, user_embedding, item_embedding,
           user_bias, item_bias, global_bias):
    uid = user_ids.astype(jnp.int32)
    iid = item_ids.astype(jnp.int32)
    ub = user_bias.reshape(-1)
    ib = item_bias.reshape(-1)
    gb16 = jnp.broadcast_to(global_bias.astype(jnp.float32), (_L,))
    return _sc_scores(uid, iid, user_embedding, item_embedding, ub, ib, gb16)


# speedup vs baseline: 2.0209x; 2.0209x over previous
import functools

import jax
import jax.numpy as jnp
from jax import lax
from jax.experimental import pallas as pl
from jax.experimental.pallas import tpu as pltpu
from jax.experimental.pallas import tpu_sc as plsc

LATENT = 64
BATCH = 16384
LANES = 128

_info = plsc.get_sparse_core_info()
_NC = _info.num_cores
_NS = _info.num_subcores
_L = _info.num_lanes
_NW = _NC * _NS
_BPW = BATCH // _NW
_CHUNK = 128
_NCHUNK = _BPW // _CHUNK
_NGROUP = _BPW // _L

_mesh = plsc.VectorSubcoreMesh(core_axis_name="c", subcore_axis_name="s")


@functools.partial(
    pl.kernel,
    out_type=jax.ShapeDtypeStruct((BATCH,), jnp.float32),
    mesh=_mesh,
    compiler_params=pltpu.CompilerParams(
        needs_layout_passes=False, use_tc_tiling_on_sc=True),
    scratch_types=[
        pltpu.VMEM((_BPW,), jnp.int32),
        pltpu.VMEM((_BPW,), jnp.int32),
        pltpu.VMEM((2, LATENT, LANES), jnp.float32),
        pltpu.VMEM((2, LATENT, LANES), jnp.float32),
        pltpu.VMEM((_L, LATENT), jnp.float32),
        pltpu.VMEM((_L, LATENT), jnp.float32),
        pltpu.VMEM((_BPW,), jnp.float32),
        pltpu.VMEM((_BPW,), jnp.float32),
        pltpu.VMEM((_L,), jnp.float32),
        pltpu.VMEM((_BPW,), jnp.float32),
        pltpu.SemaphoreType.DMA,
    ],
)
def _sc_scores(uid_hbm, iid_hbm, ut_hbm, it_hbm, ubias_hbm, ibias_hbm,
               gb_hbm, out_hbm,
               uid_v, iid_v, ublk_v, iblk_v, ucmp_v, icmp_v,
               ubias_v, ibias_v, gb_v, out_v, sem):
    wid = lax.axis_index("s") * _NC + lax.axis_index("c")
    base = wid * _BPW

    pltpu.sync_copy(uid_hbm.at[pl.ds(base, _BPW)], uid_v)
    pltpu.sync_copy(iid_hbm.at[pl.ds(base, _BPW)], iid_v)
    pltpu.sync_copy(gb_hbm, gb_v)

    bias_copies = []
    for k in range(_NCHUNK):
        sl = pl.ds(k * _CHUNK, _CHUNK)
        bias_copies.append(pltpu.async_copy(
            ubias_hbm.at[uid_v.at[sl]], ubias_v.at[sl], sem))
        bias_copies.append(pltpu.async_copy(
            ibias_hbm.at[iid_v.at[sl]], ibias_v.at[sl], sem))
    for c in bias_copies:
        c.wait()

    gb = gb_v[...]
    lane = lax.iota(jnp.int32, _L)
    zero16 = jnp.zeros((_L,), jnp.int32)

    def _enqueue(tab_hbm, blk_v, r, slot):
        s0 = (r // LANES) * LANES
        return pltpu.async_copy(
            tab_hbm.at[:, pl.ds(s0, LANES)], blk_v.at[slot], sem)

    def _extract(blk_v, cmp_v, r, slot, j):
        col = zero16 + (r % LANES)
        slot_v = zero16 + slot
        for c in range(LATENT // _L):
            dl = c * _L + lane
            vals = plsc.load_gather(blk_v, [slot_v, dl, col])
            plsc.store_scatter(cmp_v, [zero16 + j, dl], vals)

    def group(g, carry):
        ruv = uid_v[pl.ds(g * _L, _L)]
        riv = iid_v[pl.ds(g * _L, _L)]

        pending = (_enqueue(ut_hbm, ublk_v, ruv[0], 0),
                   _enqueue(it_hbm, iblk_v, riv[0], 0))
        for j in range(_L):
            if j + 1 < _L:
                nxt = (_enqueue(ut_hbm, ublk_v, ruv[j + 1], (j + 1) % 2),
                       _enqueue(it_hbm, iblk_v, riv[j + 1], (j + 1) % 2))
            pending[0].wait()
            pending[1].wait()
            _extract(ublk_v, ucmp_v, ruv[j], j % 2, j)
            _extract(iblk_v, icmp_v, riv[j], j % 2, j)
            if j + 1 < _L:
                pending = nxt

        accs = [jnp.zeros((_L,), jnp.float32) for _ in range(4)]
        for d in range(LATENT):
            dsplat = zero16 + d
            gu = plsc.load_gather(ucmp_v, [lane, dsplat])
            gi = plsc.load_gather(icmp_v, [lane, dsplat])
            accs[d % 4] = accs[d % 4] + gu * gi
        res = (accs[0] + accs[1]) + (accs[2] + accs[3])
        res = res + ubias_v[pl.ds(g * _L, _L)] + ibias_v[pl.ds(g * _L, _L)] + gb
        out_v[pl.ds(g * _L, _L)] = res
        return carry

    lax.fori_loop(0, _NGROUP, group, 0)

    pltpu.sync_copy(out_v, out_hbm.at[pl.ds(base, _BPW)])


def kernel(user_ids, item_ids, user_embedding, item_embedding,
           user_bias, item_bias, global_bias):
    uid = user_ids.astype(jnp.int32)
    iid = item_ids.astype(jnp.int32)
    ub = user_bias.reshape(-1)
    ib = item_bias.reshape(-1)
    gb16 = jnp.broadcast_to(global_bias.astype(jnp.float32), (_L,))
    return _sc_scores(uid, iid, user_embedding.T, item_embedding.T,
                      ub, ib, gb16)
